# baseline (speedup 1.0000x reference)
import dataclasses
import functools

import jax
import jax.numpy as jnp
from jax import lax
from jax.experimental import pallas as pl
from jax.experimental.pallas import tpu as pltpu
from jax.experimental.pallas import tpu_sc as plsc

N = 10000
E = 320000
D = 128
H = 8
HD = 16
L = 16
NC = 2
NS = 16
NW = NC * NS
EPW = E // NW
W = 40
NWIN = EPW // W
MROW = 144
RPT = N // NS


def _proj_body(x_ref, ws_ref, wr_ref, bs_ref, br_ref, xs_ref, xr_ref):
    x = x_ref[...]
    xs_ref[...] = lax.dot_general(
        x, ws_ref[...], (((1,), (0,)), ((), ())),
        precision=lax.Precision.HIGHEST,
        preferred_element_type=jnp.float32) + bs_ref[...]
    xr_ref[...] = lax.dot_general(
        x, wr_ref[...], (((1,), (0,)), ((), ())),
        precision=lax.Precision.HIGHEST,
        preferred_element_type=jnp.float32) + br_ref[...]


def _project(x, ws, wr, bs, br):
    blk = 1000
    grid = N // blk
    return pl.pallas_call(
        _proj_body,
        grid=(grid,),
        in_specs=[
            pl.BlockSpec((blk, D), lambda i: (i, 0)),
            pl.BlockSpec((D, D), lambda i: (0, 0)),
            pl.BlockSpec((D, D), lambda i: (0, 0)),
            pl.BlockSpec((1, D), lambda i: (0, 0)),
            pl.BlockSpec((1, D), lambda i: (0, 0)),
        ],
        out_specs=[
            pl.BlockSpec((blk, D), lambda i: (i, 0)),
            pl.BlockSpec((blk, D), lambda i: (i, 0)),
        ],
        out_shape=[
            jax.ShapeDtypeStruct((N, D), jnp.float32),
            jax.ShapeDtypeStruct((N, D), jnp.float32),
        ],
    )(x, ws, wr, bs, br)


def _sc_body(xs_hbm, xr_hbm, snd_hbm, rcv_hbm, av_hbm, zz_hbm, out_hbm,
             agg_sh, idx_s0, idx_r0, idx_c0, s_v0, r_v0, m_v0,
             idx_s1, idx_r1, idx_c1, s_v1, r_v1, m_v1, a_v,
             si_s0, si_r0, si_s1, si_r1, si_c0, si_c1,
             sg_s0, sg_r0, sg_s1, sg_r1, sc0, sc1):
    cid = lax.axis_index("c")
    sid = lax.axis_index("s")
    wid = sid * NC + cid

    idx_s = (idx_s0, idx_s1)
    idx_r = (idx_r0, idx_r1)
    idx_c = (idx_c0, idx_c1)
    s_v = (s_v0, s_v1)
    r_v = (r_v0, r_v1)
    m_v = (m_v0, m_v1)
    si_s = (si_s0, si_s1)
    si_r = (si_r0, si_r1)
    si_c = (si_c0, si_c1)
    sg_s = (sg_s0, sg_s1)
    sg_r = (sg_r0, sg_r1)
    sc = (sc0, sc1)

    pltpu.sync_copy(zz_hbm.at[pl.ds(sid * RPT, RPT)],
                    agg_sh.at[pl.ds(sid * RPT, RPT)])

    pltpu.sync_copy(av_hbm, a_v)
    plsc.subcore_barrier()

    a_vec = a_v[...]
    iota = lax.iota(jnp.int32, L)
    head_mask = [iota == h for h in range(H)]

    def issue_idx(win, b):
        e0 = wid * EPW + win * W
        pltpu.async_copy(snd_hbm.at[pl.ds(e0, W)], idx_s[b], si_s[b])
        pltpu.async_copy(rcv_hbm.at[pl.ds(e0, W)], idx_r[b], si_r[b])

    def wait_idx(b):
        pltpu.make_async_copy(snd_hbm.at[pl.ds(0, W)], idx_s[b], si_s[b]).wait()
        pltpu.make_async_copy(rcv_hbm.at[pl.ds(0, W)], idx_r[b], si_r[b]).wait()

    def issue_gather(b):
        pltpu.async_copy(xs_hbm.at[idx_s[b]], s_v[b], sg_s[b])
        pltpu.async_copy(xr_hbm.at[idx_r[b]], r_v[b], sg_r[b])

    def wait_gather(b):
        pltpu.make_async_copy(xs_hbm.at[idx_s[b]], s_v[b], sg_s[b]).wait()
        pltpu.make_async_copy(xr_hbm.at[idx_r[b]], r_v[b], sg_r[b]).wait()

    def wait_scatter(b):
        pltpu.make_async_copy(m_v[b], agg_sh.at[idx_c[b]], sc[b]).wait()

    e0 = wid * EPW
    pltpu.sync_copy(snd_hbm.at[pl.ds(e0, W)], idx_s[0])
    pltpu.sync_copy(rcv_hbm.at[pl.ds(e0, W)], idx_r[0])
    issue_gather(0)
    issue_idx(1, 1)

    @pl.loop(0, NWIN, step=2)
    def _(base):
        for b in (0, 1):
            win = base + b
            nb = 1 - b
            wait_idx(nb)
            issue_gather(nb)
            wait_gather(b)
            @pl.when(win >= 2)
            def _():
                wait_scatter(b)
            ec = wid * EPW + win * W
            pltpu.async_copy(rcv_hbm.at[pl.ds(ec, W)], idx_c[b], si_c[b])
            nxt = jnp.minimum(win + 2, NWIN - 1)
            issue_idx(nxt, b)

            @plsc.parallel_loop(0, W, unroll=8)
            def _(w):
                for h in range(H):
                    s_h = s_v[b][w, pl.ds(h * HD, HD)]
                    r_h = r_v[b][w, pl.ds(h * HD, HD)]
                    z = s_h + r_h
                    u = jnp.exp(jnp.minimum(z, 20.0))
                    t = u * (u + 2.0)
                    r_v[b][w, pl.ds(h * HD, HD)] = (z * a_vec) * t / (t + 2.0)

            @plsc.parallel_loop(0, W, unroll=4)
            def _(w):
                du = jnp.zeros((L,), jnp.float32)
                for h in range(H):
                    p_h = r_v[b][w, pl.ds(h * HD, HD)]
                    uv = jnp.exp(lax.broadcast(jnp.sum(p_h), (L,)))
                    s_h = s_v[b][w, pl.ds(h * HD, HD)]
                    m_v[b][w, pl.ds(h * HD, HD)] = s_h * uv
                    du = du + jnp.where(head_mask[h], uv, 0.0)
                m_v[b][w, pl.ds(D, L)] = du

            pltpu.make_async_copy(rcv_hbm.at[pl.ds(0, W)], idx_c[b],
                                  si_c[b]).wait()
            pltpu.async_copy(m_v[b], agg_sh.at[idx_c[b]], sc[b], add=True)

    wait_scatter(0)
    wait_scatter(1)
    wait_gather(0)
    wait_idx(1)

    plsc.subcore_barrier()

    pltpu.sync_copy(agg_sh.at[pl.ds(sid * RPT, RPT)],
                    out_hbm.at[cid, pl.ds(sid * RPT, RPT)])


def _sc_edge_pass(xs, xr, snd, rcv, a_vec):
    mesh = plsc.VectorSubcoreMesh(core_axis_name="c", subcore_axis_name="s")
    cp = pltpu.CompilerParams()
    if "needs_layout_passes" in pltpu.CompilerParams.__dataclass_fields__:
        cp = dataclasses.replace(cp, needs_layout_passes=False)
    if "use_tc_tiling_on_sc" in pltpu.CompilerParams.__dataclass_fields__:
        cp = dataclasses.replace(cp, use_tc_tiling_on_sc=False)
    kern = pl.kernel(
        _sc_body,
        compiler_params=cp,
        out_type=jax.ShapeDtypeStruct((NC, N, MROW), jnp.float32),
        mesh=mesh,
        scratch_types=(
            [pltpu.VMEM_SHARED((N, MROW), jnp.float32)]
            + 2 * [
                pltpu.VMEM((W,), jnp.int32),
                pltpu.VMEM((W,), jnp.int32),
                pltpu.VMEM((W,), jnp.int32),
                pltpu.VMEM((W, D), jnp.float32),
                pltpu.VMEM((W, D), jnp.float32),
                pltpu.VMEM((W, MROW), jnp.float32),
            ]
            + [
                pltpu.VMEM((L,), jnp.float32),
            ]
            + 12 * [pltpu.SemaphoreType.DMA]
        ),
    )
    zz = jnp.zeros((N, MROW), jnp.float32)
    return kern(xs, xr, snd, rcv, a_vec, zz)


def _comb_body(p0_ref, p1_ref, o_ref):
    a = p0_ref[:, :D] + p1_ref[:, :D]
    dnm = p0_ref[:, D:D + H] + p1_ref[:, D:D + H]
    drep = jnp.concatenate(
        [jnp.broadcast_to(dnm[:, h:h + 1], (dnm.shape[0], HD))
         for h in range(H)], axis=1)
    o_ref[...] = jnp.where(drep > 0.0, a / drep, 0.0)


def _combine(partials):
    blk = 1000
    grid = N // blk
    p0 = partials[0]
    p1 = partials[1]
    return pl.pallas_call(
        _comb_body,
        grid=(grid,),
        in_specs=[
            pl.BlockSpec((blk, MROW), lambda i: (i, 0)),
            pl.BlockSpec((blk, MROW), lambda i: (i, 0)),
        ],
        out_specs=pl.BlockSpec((blk, D), lambda i: (i, 0)),
        out_shape=jax.ShapeDtypeStruct((N, D), jnp.float32),
    )(p0, p1)


def kernel(x, edge_index, Ws_k, Ws_b, Wr_k, Wr_b, A_k, A_b):
    ws = Ws_k.reshape(D, H * HD)
    wr = Wr_k.reshape(D, H * HD)
    bs = Ws_b.reshape(1, H * HD)
    br = Wr_b.reshape(1, H * HD)
    a_vec = A_k.reshape(HD)
    snd = edge_index[0]
    rcv = edge_index[1]

    xs, xr = _project(x, ws, wr, bs, br)
    partials = _sc_edge_pass(xs, xr, snd, rcv, a_vec)
    return _combine(partials)


# speedup vs baseline: 1.8053x; 1.8053x over previous
import dataclasses
import functools

import jax
import jax.numpy as jnp
from jax import lax
from jax.experimental import pallas as pl
from jax.experimental.pallas import tpu as pltpu
from jax.experimental.pallas import tpu_sc as plsc

N = 10000
E = 320000
D = 128
H = 8
HD = 16
L = 16
NC = 2
NS = 16
NW = NC * NS
EPW = E // NW
W = 40
NWIN = EPW // W
MROW = 144
RPT = N // NS


def _proj_body(x_ref, ws_ref, wr_ref, bs_ref, br_ref, xs_ref, xr_ref):
    x = x_ref[...]
    xs_ref[...] = lax.dot_general(
        x, ws_ref[...], (((1,), (0,)), ((), ())),
        precision=lax.Precision.HIGHEST,
        preferred_element_type=jnp.float32) + bs_ref[...]
    xr_ref[...] = lax.dot_general(
        x, wr_ref[...], (((1,), (0,)), ((), ())),
        precision=lax.Precision.HIGHEST,
        preferred_element_type=jnp.float32) + br_ref[...]


def _project(x, ws, wr, bs, br):
    blk = 1000
    grid = N // blk
    return pl.pallas_call(
        _proj_body,
        grid=(grid,),
        in_specs=[
            pl.BlockSpec((blk, D), lambda i: (i, 0)),
            pl.BlockSpec((D, D), lambda i: (0, 0)),
            pl.BlockSpec((D, D), lambda i: (0, 0)),
            pl.BlockSpec((1, D), lambda i: (0, 0)),
            pl.BlockSpec((1, D), lambda i: (0, 0)),
        ],
        out_specs=[
            pl.BlockSpec((blk, D), lambda i: (i, 0)),
            pl.BlockSpec((blk, D), lambda i: (i, 0)),
        ],
        out_shape=[
            jax.ShapeDtypeStruct((N, D), jnp.float32),
            jax.ShapeDtypeStruct((N, D), jnp.float32),
        ],
    )(x, ws, wr, bs, br)


def _sc_body(xs_hbm, xr_hbm, snd_hbm, rcv_hbm, av_hbm, zz_hbm, out_hbm,
             agg_sh, idx_s0, idx_r0, idx_c0, s_v0, r_v0, m_v0,
             idx_s1, idx_r1, idx_c1, s_v1, r_v1, m_v1, a_v,
             si_s0, si_r0, si_s1, si_r1, si_c0, si_c1,
             sg_s0, sg_r0, sg_s1, sg_r1, sc0, sc1):
    cid = lax.axis_index("c")
    sid = lax.axis_index("s")
    wid = sid * NC + cid

    idx_s = (idx_s0, idx_s1)
    idx_r = (idx_r0, idx_r1)
    idx_c = (idx_c0, idx_c1)
    s_v = (s_v0, s_v1)
    r_v = (r_v0, r_v1)
    m_v = (m_v0, m_v1)
    si_s = (si_s0, si_s1)
    si_r = (si_r0, si_r1)
    si_c = (si_c0, si_c1)
    sg_s = (sg_s0, sg_s1)
    sg_r = (sg_r0, sg_r1)
    sc = (sc0, sc1)

    pltpu.sync_copy(zz_hbm.at[pl.ds(sid * RPT, RPT)],
                    agg_sh.at[pl.ds(sid * RPT, RPT)])

    pltpu.sync_copy(av_hbm, a_v)
    plsc.subcore_barrier()

    a_vec = a_v[...]
    iota = lax.iota(jnp.int32, L)
    head_mask = [iota == h for h in range(H)]

    def issue_idx(win, b):
        e0 = wid * EPW + win * W
        pltpu.async_copy(snd_hbm.at[pl.ds(e0, W)], idx_s[b], si_s[b])
        pltpu.async_copy(rcv_hbm.at[pl.ds(e0, W)], idx_r[b], si_r[b])

    def wait_idx(b):
        pltpu.make_async_copy(snd_hbm.at[pl.ds(0, W)], idx_s[b], si_s[b]).wait()
        pltpu.make_async_copy(rcv_hbm.at[pl.ds(0, W)], idx_r[b], si_r[b]).wait()

    def issue_gather(b):
        pltpu.async_copy(xs_hbm.at[idx_s[b]], s_v[b], sg_s[b])
        pltpu.async_copy(xr_hbm.at[idx_r[b]], r_v[b], sg_r[b])

    def wait_gather(b):
        pltpu.make_async_copy(xs_hbm.at[idx_s[b]], s_v[b], sg_s[b]).wait()
        pltpu.make_async_copy(xr_hbm.at[idx_r[b]], r_v[b], sg_r[b]).wait()

    def wait_scatter(b):
        pltpu.make_async_copy(m_v[b], agg_sh.at[idx_c[b]], sc[b]).wait()

    e0 = wid * EPW
    pltpu.sync_copy(snd_hbm.at[pl.ds(e0, W)], idx_s[0])
    pltpu.sync_copy(rcv_hbm.at[pl.ds(e0, W)], idx_r[0])
    issue_gather(0)
    issue_idx(1, 1)

    @pl.loop(0, NWIN, step=2)
    def _(base):
        for b in (0, 1):
            win = base + b
            nb = 1 - b
            wait_idx(nb)
            issue_gather(nb)
            wait_gather(b)
            @pl.when(win >= 2)
            def _():
                wait_scatter(b)
            ec = wid * EPW + win * W
            pltpu.async_copy(rcv_hbm.at[pl.ds(ec, W)], idx_c[b], si_c[b])
            nxt = jnp.minimum(win + 2, NWIN - 1)
            issue_idx(nxt, b)

            @plsc.parallel_loop(0, W, unroll=5)
            def _(w):
                du = jnp.zeros((L,), jnp.float32)
                for h in range(H):
                    s_h = s_v[b][w, pl.ds(h * HD, HD)]
                    r_h = r_v[b][w, pl.ds(h * HD, HD)]
                    z = s_h + r_h
                    u = jnp.exp(jnp.minimum(z, 20.0))
                    t = u * (u + 2.0)
                    mish = z * t / (t + 2.0)
                    logit = jnp.sum(mish * a_vec)
                    uv = jnp.exp(lax.broadcast(logit, (L,)))
                    m_v[b][w, pl.ds(h * HD, HD)] = s_h * uv
                    du = du + jnp.where(head_mask[h], uv, 0.0)
                m_v[b][w, pl.ds(D, L)] = du

            pltpu.make_async_copy(rcv_hbm.at[pl.ds(0, W)], idx_c[b],
                                  si_c[b]).wait()
            pltpu.async_copy(m_v[b], agg_sh.at[idx_c[b]], sc[b], add=True)

    wait_scatter(0)
    wait_scatter(1)
    wait_gather(0)
    wait_idx(1)

    plsc.subcore_barrier()

    pltpu.sync_copy(agg_sh.at[pl.ds(sid * RPT, RPT)],
                    out_hbm.at[cid, pl.ds(sid * RPT, RPT)])


def _sc_edge_pass(xs, xr, snd, rcv, a_vec):
    mesh = plsc.VectorSubcoreMesh(core_axis_name="c", subcore_axis_name="s")
    cp = pltpu.CompilerParams()
    if "needs_layout_passes" in pltpu.CompilerParams.__dataclass_fields__:
        cp = dataclasses.replace(cp, needs_layout_passes=False)
    if "use_tc_tiling_on_sc" in pltpu.CompilerParams.__dataclass_fields__:
        cp = dataclasses.replace(cp, use_tc_tiling_on_sc=False)
    kern = pl.kernel(
        _sc_body,
        compiler_params=cp,
        out_type=jax.ShapeDtypeStruct((NC, N, MROW), jnp.float32),
        mesh=mesh,
        scratch_types=(
            [pltpu.VMEM_SHARED((N, MROW), jnp.float32)]
            + 2 * [
                pltpu.VMEM((W,), jnp.int32),
                pltpu.VMEM((W,), jnp.int32),
                pltpu.VMEM((W,), jnp.int32),
                pltpu.VMEM((W, D), jnp.float32),
                pltpu.VMEM((W, D), jnp.float32),
                pltpu.VMEM((W, MROW), jnp.float32),
            ]
            + [
                pltpu.VMEM((L,), jnp.float32),
            ]
            + 12 * [pltpu.SemaphoreType.DMA]
        ),
    )
    zz = jnp.zeros((N, MROW), jnp.float32)
    return kern(xs, xr, snd, rcv, a_vec, zz)


def _comb_body(p0_ref, p1_ref, o_ref):
    a = p0_ref[:, :D] + p1_ref[:, :D]
    dnm = p0_ref[:, D:D + H] + p1_ref[:, D:D + H]
    drep = jnp.concatenate(
        [jnp.broadcast_to(dnm[:, h:h + 1], (dnm.shape[0], HD))
         for h in range(H)], axis=1)
    o_ref[...] = jnp.where(drep > 0.0, a / drep, 0.0)


def _combine(partials):
    blk = 1000
    grid = N // blk
    p0 = partials[0]
    p1 = partials[1]
    return pl.pallas_call(
        _comb_body,
        grid=(grid,),
        in_specs=[
            pl.BlockSpec((blk, MROW), lambda i: (i, 0)),
            pl.BlockSpec((blk, MROW), lambda i: (i, 0)),
        ],
        out_specs=pl.BlockSpec((blk, D), lambda i: (i, 0)),
        out_shape=jax.ShapeDtypeStruct((N, D), jnp.float32),
    )(p0, p1)


def kernel(x, edge_index, Ws_k, Ws_b, Wr_k, Wr_b, A_k, A_b):
    ws = Ws_k.reshape(D, H * HD)
    wr = Wr_k.reshape(D, H * HD)
    bs = Ws_b.reshape(1, H * HD)
    br = Wr_b.reshape(1, H * HD)
    a_vec = A_k.reshape(HD)
    snd = edge_index[0]
    rcv = edge_index[1]

    xs, xr = _project(x, ws, wr, bs, br)
    partials = _sc_edge_pass(xs, xr, snd, rcv, a_vec)
    return _combine(partials)


# speedup vs baseline: 4.0197x; 2.2266x over previous
import dataclasses
import functools

import jax
import jax.numpy as jnp
from jax import lax
from jax.experimental import pallas as pl
from jax.experimental.pallas import tpu as pltpu
from jax.experimental.pallas import tpu_sc as plsc

N = 10000
E = 320000
D = 128
H = 8
HD = 16
L = 16
NC = 2
NS = 16
NW = NC * NS
EPW = E // NW
W = 40
NWIN = EPW // W
MROW = 144
RPT = N // NS


def _proj_body(x_ref, ws_ref, wr_ref, bs_ref, br_ref, xs_ref, xr_ref):
    x = x_ref[...]
    xs_ref[...] = lax.dot_general(
        x, ws_ref[...], (((1,), (0,)), ((), ())),
        precision=lax.Precision.HIGHEST,
        preferred_element_type=jnp.float32) + bs_ref[...]
    xr_ref[...] = lax.dot_general(
        x, wr_ref[...], (((1,), (0,)), ((), ())),
        precision=lax.Precision.HIGHEST,
        preferred_element_type=jnp.float32) + br_ref[...]


def _project(x, ws, wr, bs, br):
    blk = 1000
    grid = N // blk
    return pl.pallas_call(
        _proj_body,
        grid=(grid,),
        in_specs=[
            pl.BlockSpec((blk, D), lambda i: (i, 0)),
            pl.BlockSpec((D, D), lambda i: (0, 0)),
            pl.BlockSpec((D, D), lambda i: (0, 0)),
            pl.BlockSpec((1, D), lambda i: (0, 0)),
            pl.BlockSpec((1, D), lambda i: (0, 0)),
        ],
        out_specs=[
            pl.BlockSpec((blk, D), lambda i: (i, 0)),
            pl.BlockSpec((blk, D), lambda i: (i, 0)),
        ],
        out_shape=[
            jax.ShapeDtypeStruct((N, D), jnp.float32),
            jax.ShapeDtypeStruct((N, D), jnp.float32),
        ],
    )(x, ws, wr, bs, br)


def _sc_body(xs_hbm, xr_hbm, snd_hbm, rcv_hbm, av_hbm, zz_hbm, out_hbm,
             agg_sh, idx_s0, idx_r0, idx_c0, s_v0, r_v0, m_v0,
             idx_s1, idx_r1, idx_c1, s_v1, r_v1, m_v1, a_v,
             si_s0, si_r0, si_s1, si_r1, si_c0, si_c1,
             sg_s0, sg_r0, sg_s1, sg_r1, sc0, sc1):
    cid = lax.axis_index("c")
    sid = lax.axis_index("s")
    wid = sid * NC + cid

    idx_s = (idx_s0, idx_s1)
    idx_r = (idx_r0, idx_r1)
    idx_c = (idx_c0, idx_c1)
    s_v = (s_v0, s_v1)
    r_v = (r_v0, r_v1)
    m_v = (m_v0, m_v1)
    si_s = (si_s0, si_s1)
    si_r = (si_r0, si_r1)
    si_c = (si_c0, si_c1)
    sg_s = (sg_s0, sg_s1)
    sg_r = (sg_r0, sg_r1)
    sc = (sc0, sc1)

    pltpu.sync_copy(zz_hbm.at[pl.ds(sid * RPT, RPT)],
                    agg_sh.at[pl.ds(sid * RPT, RPT)])

    pltpu.sync_copy(av_hbm, a_v)
    plsc.subcore_barrier()

    a_vec = a_v[...]
    iota = lax.iota(jnp.int32, L)
    head_mask = [iota == h for h in range(H)]

    def issue_idx(win, b):
        e0 = wid * EPW + win * W
        pltpu.async_copy(snd_hbm.at[pl.ds(e0, W)], idx_s[b], si_s[b])
        pltpu.async_copy(rcv_hbm.at[pl.ds(e0, W)], idx_r[b], si_r[b])

    def wait_idx(b):
        pltpu.make_async_copy(snd_hbm.at[pl.ds(0, W)], idx_s[b], si_s[b]).wait()
        pltpu.make_async_copy(rcv_hbm.at[pl.ds(0, W)], idx_r[b], si_r[b]).wait()

    def issue_gather(b):
        pltpu.async_copy(xs_hbm.at[idx_s[b]], s_v[b], sg_s[b])
        pltpu.async_copy(xr_hbm.at[idx_r[b]], r_v[b], sg_r[b])

    def wait_gather(b):
        pltpu.make_async_copy(xs_hbm.at[idx_s[b]], s_v[b], sg_s[b]).wait()
        pltpu.make_async_copy(xr_hbm.at[idx_r[b]], r_v[b], sg_r[b]).wait()

    def wait_scatter(b):
        pltpu.make_async_copy(m_v[b], agg_sh.at[idx_c[b]], sc[b]).wait()

    e0 = wid * EPW
    pltpu.sync_copy(snd_hbm.at[pl.ds(e0, W)], idx_s[0])
    pltpu.sync_copy(rcv_hbm.at[pl.ds(e0, W)], idx_r[0])
    issue_gather(0)
    issue_idx(1, 1)

    @pl.loop(0, NWIN, step=2)
    def _(base):
        for b in (0, 1):
            win = base + b
            nb = 1 - b
            wait_idx(nb)
            issue_gather(nb)
            wait_gather(b)
            @pl.when(win >= 2)
            def _():
                wait_scatter(b)
            ec = wid * EPW + win * W
            pltpu.async_copy(rcv_hbm.at[pl.ds(ec, W)], idx_c[b], si_c[b])
            nxt = jnp.minimum(win + 2, NWIN - 1)
            issue_idx(nxt, b)

            @plsc.parallel_loop(0, W, unroll=4)
            def _(w):
                ss = [s_v[b][w, pl.ds(h * HD, HD)] for h in range(H)]
                rr = [r_v[b][w, pl.ds(h * HD, HD)] for h in range(H)]
                zz_ = [ss[h] + rr[h] for h in range(H)]
                uu = [jnp.exp(jnp.minimum(z, 20.0)) for z in zz_]
                tt = [u * (u + 2.0) for u in uu]
                pp = [(zz_[h] * a_vec) * tt[h] / (tt[h] + 2.0)
                      for h in range(H)]
                lg = [jnp.sum(p) for p in pp]
                uv = [jnp.exp(lax.broadcast(g, (L,))) for g in lg]
                du = jnp.zeros((L,), jnp.float32)
                for h in range(H):
                    m_v[b][w, pl.ds(h * HD, HD)] = ss[h] * uv[h]
                    du = du + jnp.where(head_mask[h], uv[h], 0.0)
                m_v[b][w, pl.ds(D, L)] = du

            pltpu.make_async_copy(rcv_hbm.at[pl.ds(0, W)], idx_c[b],
                                  si_c[b]).wait()
            pltpu.async_copy(m_v[b], agg_sh.at[idx_c[b]], sc[b], add=True)

    wait_scatter(0)
    wait_scatter(1)
    wait_gather(0)
    wait_idx(1)

    plsc.subcore_barrier()

    pltpu.sync_copy(agg_sh.at[pl.ds(sid * RPT, RPT)],
                    out_hbm.at[cid, pl.ds(sid * RPT, RPT)])


def _sc_edge_pass(xs, xr, snd, rcv, a_vec):
    mesh = plsc.VectorSubcoreMesh(core_axis_name="c", subcore_axis_name="s")
    cp = pltpu.CompilerParams()
    if "needs_layout_passes" in pltpu.CompilerParams.__dataclass_fields__:
        cp = dataclasses.replace(cp, needs_layout_passes=False)
    if "use_tc_tiling_on_sc" in pltpu.CompilerParams.__dataclass_fields__:
        cp = dataclasses.replace(cp, use_tc_tiling_on_sc=False)
    kern = pl.kernel(
        _sc_body,
        compiler_params=cp,
        out_type=jax.ShapeDtypeStruct((NC, N, MROW), jnp.float32),
        mesh=mesh,
        scratch_types=(
            [pltpu.VMEM_SHARED((N, MROW), jnp.float32)]
            + 2 * [
                pltpu.VMEM((W,), jnp.int32),
                pltpu.VMEM((W,), jnp.int32),
                pltpu.VMEM((W,), jnp.int32),
                pltpu.VMEM((W, D), jnp.float32),
                pltpu.VMEM((W, D), jnp.float32),
                pltpu.VMEM((W, MROW), jnp.float32),
            ]
            + [
                pltpu.VMEM((L,), jnp.float32),
            ]
            + 12 * [pltpu.SemaphoreType.DMA]
        ),
    )
    zz = jnp.zeros((N, MROW), jnp.float32)
    return kern(xs, xr, snd, rcv, a_vec, zz)


def _comb_body(p0_ref, p1_ref, o_ref):
    a = p0_ref[:, :D] + p1_ref[:, :D]
    dnm = p0_ref[:, D:D + H] + p1_ref[:, D:D + H]
    drep = jnp.concatenate(
        [jnp.broadcast_to(dnm[:, h:h + 1], (dnm.shape[0], HD))
         for h in range(H)], axis=1)
    o_ref[...] = jnp.where(drep > 0.0, a / drep, 0.0)


def _combine(partials):
    blk = 1000
    grid = N // blk
    p0 = partials[0]
    p1 = partials[1]
    return pl.pallas_call(
        _comb_body,
        grid=(grid,),
        in_specs=[
            pl.BlockSpec((blk, MROW), lambda i: (i, 0)),
            pl.BlockSpec((blk, MROW), lambda i: (i, 0)),
        ],
        out_specs=pl.BlockSpec((blk, D), lambda i: (i, 0)),
        out_shape=jax.ShapeDtypeStruct((N, D), jnp.float32),
    )(p0, p1)


def kernel(x, edge_index, Ws_k, Ws_b, Wr_k, Wr_b, A_k, A_b):
    ws = Ws_k.reshape(D, H * HD)
    wr = Wr_k.reshape(D, H * HD)
    bs = Ws_b.reshape(1, H * HD)
    br = Wr_b.reshape(1, H * HD)
    a_vec = A_k.reshape(HD)
    snd = edge_index[0]
    rcv = edge_index[1]

    xs, xr = _project(x, ws, wr, bs, br)
    partials = _sc_edge_pass(xs, xr, snd, rcv, a_vec)
    return _combine(partials)


# speedup vs baseline: 4.6946x; 1.1679x over previous
import dataclasses
import functools

import jax
import jax.numpy as jnp
from jax import lax
from jax.experimental import pallas as pl
from jax.experimental.pallas import tpu as pltpu
from jax.experimental.pallas import tpu_sc as plsc

N = 10000
E = 320000
D = 128
H = 8
HD = 16
L = 16
NC = 2
NS = 16
NW = NC * NS
EPW = E // NW
W = 40
NWIN = EPW // W
MROW = 144
RPT = N // NS


def _proj_body(x_ref, ws_ref, wr_ref, bs_ref, br_ref, xs_ref, xr_ref):
    x = x_ref[...]
    xs_ref[...] = lax.dot_general(
        x, ws_ref[...], (((1,), (0,)), ((), ())),
        precision=lax.Precision.HIGHEST,
        preferred_element_type=jnp.float32) + bs_ref[...]
    xr_ref[...] = lax.dot_general(
        x, wr_ref[...], (((1,), (0,)), ((), ())),
        precision=lax.Precision.HIGHEST,
        preferred_element_type=jnp.float32) + br_ref[...]


def _project(x, ws, wr, bs, br):
    blk = 1000
    grid = N // blk
    return pl.pallas_call(
        _proj_body,
        grid=(grid,),
        in_specs=[
            pl.BlockSpec((blk, D), lambda i: (i, 0)),
            pl.BlockSpec((D, D), lambda i: (0, 0)),
            pl.BlockSpec((D, D), lambda i: (0, 0)),
            pl.BlockSpec((1, D), lambda i: (0, 0)),
            pl.BlockSpec((1, D), lambda i: (0, 0)),
        ],
        out_specs=[
            pl.BlockSpec((blk, D), lambda i: (i, 0)),
            pl.BlockSpec((blk, D), lambda i: (i, 0)),
        ],
        out_shape=[
            jax.ShapeDtypeStruct((N, D), jnp.float32),
            jax.ShapeDtypeStruct((N, D), jnp.float32),
        ],
    )(x, ws, wr, bs, br)


def _sc_body(xs_hbm, xr_hbm, snd_hbm, rcv_hbm, av_hbm, zz_hbm, out_hbm,
             agg_sh, idx_s0, idx_r0, idx_c0, s_v0, r_v0, m_v0,
             idx_s1, idx_r1, idx_c1, s_v1, r_v1, m_v1, a_v,
             si_s0, si_r0, si_s1, si_r1, si_c0, si_c1,
             sg_s0, sg_r0, sg_s1, sg_r1, sc0, sc1):
    cid = lax.axis_index("c")
    sid = lax.axis_index("s")
    wid = sid * NC + cid

    idx_s = (idx_s0, idx_s1)
    idx_r = (idx_r0, idx_r1)
    idx_c = (idx_c0, idx_c1)
    s_v = (s_v0, s_v1)
    r_v = (r_v0, r_v1)
    m_v = (m_v0, m_v1)
    si_s = (si_s0, si_s1)
    si_r = (si_r0, si_r1)
    si_c = (si_c0, si_c1)
    sg_s = (sg_s0, sg_s1)
    sg_r = (sg_r0, sg_r1)
    sc = (sc0, sc1)

    pltpu.sync_copy(zz_hbm.at[pl.ds(sid * RPT, RPT)],
                    agg_sh.at[pl.ds(sid * RPT, RPT)])

    pltpu.sync_copy(av_hbm, a_v)
    plsc.subcore_barrier()

    a_vec = a_v[...]
    iota = lax.iota(jnp.int32, L)
    head_mask = [iota == h for h in range(H)]

    def issue_idx(win, b):
        e0 = wid * EPW + win * W
        pltpu.async_copy(snd_hbm.at[pl.ds(e0, W)], idx_s[b], si_s[b])
        pltpu.async_copy(rcv_hbm.at[pl.ds(e0, W)], idx_r[b], si_r[b])

    def wait_idx(b):
        pltpu.make_async_copy(snd_hbm.at[pl.ds(0, W)], idx_s[b], si_s[b]).wait()
        pltpu.make_async_copy(rcv_hbm.at[pl.ds(0, W)], idx_r[b], si_r[b]).wait()

    def issue_gather(b):
        pltpu.async_copy(xs_hbm.at[idx_s[b]], s_v[b], sg_s[b])
        pltpu.async_copy(xr_hbm.at[idx_r[b]], r_v[b], sg_r[b])

    def wait_gather(b):
        pltpu.make_async_copy(xs_hbm.at[idx_s[b]], s_v[b], sg_s[b]).wait()
        pltpu.make_async_copy(xr_hbm.at[idx_r[b]], r_v[b], sg_r[b]).wait()

    def wait_scatter(b):
        pltpu.make_async_copy(m_v[b], agg_sh.at[idx_c[b]], sc[b]).wait()

    e0 = wid * EPW
    pltpu.sync_copy(snd_hbm.at[pl.ds(e0, W)], idx_s[0])
    pltpu.sync_copy(rcv_hbm.at[pl.ds(e0, W)], idx_r[0])
    issue_gather(0)
    issue_idx(1, 1)

    @pl.loop(0, NWIN, step=2)
    def _(base):
        for b in (0, 1):
            win = base + b
            nb = 1 - b
            wait_idx(nb)
            issue_gather(nb)
            wait_gather(b)
            @pl.when(win >= 2)
            def _():
                wait_scatter(b)
            ec = wid * EPW + win * W
            pltpu.async_copy(rcv_hbm.at[pl.ds(ec, W)], idx_c[b], si_c[b])
            nxt = jnp.minimum(win + 2, NWIN - 1)
            issue_idx(nxt, b)

            @plsc.parallel_loop(0, W, unroll=1)
            def _(w):
                ss = [s_v[b][w, pl.ds(h * HD, HD)] for h in range(H)]
                rr = [r_v[b][w, pl.ds(h * HD, HD)] for h in range(H)]
                zz_ = [ss[h] + rr[h] for h in range(H)]
                uu = [jnp.exp(jnp.minimum(z, 20.0)) for z in zz_]
                tt = [u * (u + 2.0) for u in uu]
                pp = [(zz_[h] * a_vec) * tt[h] / (tt[h] + 2.0)
                      for h in range(H)]
                lg = [jnp.sum(p) for p in pp]
                uv = [jnp.exp(lax.broadcast(g, (L,))) for g in lg]
                du = jnp.zeros((L,), jnp.float32)
                for h in range(H):
                    m_v[b][w, pl.ds(h * HD, HD)] = ss[h] * uv[h]
                    du = du + jnp.where(head_mask[h], uv[h], 0.0)
                m_v[b][w, pl.ds(D, L)] = du

            pltpu.make_async_copy(rcv_hbm.at[pl.ds(0, W)], idx_c[b],
                                  si_c[b]).wait()
            pltpu.async_copy(m_v[b], agg_sh.at[idx_c[b]], sc[b], add=True)

    wait_scatter(0)
    wait_scatter(1)
    wait_gather(0)
    wait_idx(1)

    plsc.subcore_barrier()

    pltpu.sync_copy(agg_sh.at[pl.ds(sid * RPT, RPT)],
                    out_hbm.at[cid, pl.ds(sid * RPT, RPT)])


def _sc_edge_pass(xs, xr, snd, rcv, a_vec):
    mesh = plsc.VectorSubcoreMesh(core_axis_name="c", subcore_axis_name="s")
    cp = pltpu.CompilerParams()
    if "needs_layout_passes" in pltpu.CompilerParams.__dataclass_fields__:
        cp = dataclasses.replace(cp, needs_layout_passes=False)
    if "use_tc_tiling_on_sc" in pltpu.CompilerParams.__dataclass_fields__:
        cp = dataclasses.replace(cp, use_tc_tiling_on_sc=False)
    kern = pl.kernel(
        _sc_body,
        compiler_params=cp,
        out_type=jax.ShapeDtypeStruct((NC, N, MROW), jnp.float32),
        mesh=mesh,
        scratch_types=(
            [pltpu.VMEM_SHARED((N, MROW), jnp.float32)]
            + 2 * [
                pltpu.VMEM((W,), jnp.int32),
                pltpu.VMEM((W,), jnp.int32),
                pltpu.VMEM((W,), jnp.int32),
                pltpu.VMEM((W, D), jnp.float32),
                pltpu.VMEM((W, D), jnp.float32),
                pltpu.VMEM((W, MROW), jnp.float32),
            ]
            + [
                pltpu.VMEM((L,), jnp.float32),
            ]
            + 12 * [pltpu.SemaphoreType.DMA]
        ),
    )
    zz = jnp.zeros((N, MROW), jnp.float32)
    return kern(xs, xr, snd, rcv, a_vec, zz)


def _comb_body(p0_ref, p1_ref, o_ref):
    a = p0_ref[:, :D] + p1_ref[:, :D]
    dnm = p0_ref[:, D:D + H] + p1_ref[:, D:D + H]
    drep = jnp.concatenate(
        [jnp.broadcast_to(dnm[:, h:h + 1], (dnm.shape[0], HD))
         for h in range(H)], axis=1)
    o_ref[...] = jnp.where(drep > 0.0, a / drep, 0.0)


def _combine(partials):
    blk = 1000
    grid = N // blk
    p0 = partials[0]
    p1 = partials[1]
    return pl.pallas_call(
        _comb_body,
        grid=(grid,),
        in_specs=[
            pl.BlockSpec((blk, MROW), lambda i: (i, 0)),
            pl.BlockSpec((blk, MROW), lambda i: (i, 0)),
        ],
        out_specs=pl.BlockSpec((blk, D), lambda i: (i, 0)),
        out_shape=jax.ShapeDtypeStruct((N, D), jnp.float32),
    )(p0, p1)


def kernel(x, edge_index, Ws_k, Ws_b, Wr_k, Wr_b, A_k, A_b):
    ws = Ws_k.reshape(D, H * HD)
    wr = Wr_k.reshape(D, H * HD)
    bs = Ws_b.reshape(1, H * HD)
    br = Wr_b.reshape(1, H * HD)
    a_vec = A_k.reshape(HD)
    snd = edge_index[0]
    rcv = edge_index[1]

    xs, xr = _project(x, ws, wr, bs, br)
    partials = _sc_edge_pass(xs, xr, snd, rcv, a_vec)
    return _combine(partials)


# speedup vs baseline: 4.8389x; 1.0307x over previous
import dataclasses
import functools

import jax
import jax.numpy as jnp
from jax import lax
from jax.experimental import pallas as pl
from jax.experimental.pallas import tpu as pltpu
from jax.experimental.pallas import tpu_sc as plsc

N = 10000
E = 320000
D = 128
H = 8
HD = 16
L = 16
NC = 2
NS = 16
NW = NC * NS
EPW = E // NW
W = 40
NWIN = EPW // W
MROW = 144
RPT = N // NS


def _proj_body(x_ref, ws_ref, wr_ref, bs_ref, br_ref, xs_ref, xr_ref):
    x = x_ref[...]
    xs_ref[...] = lax.dot_general(
        x, ws_ref[...], (((1,), (0,)), ((), ())),
        precision=lax.Precision.HIGHEST,
        preferred_element_type=jnp.float32) + bs_ref[...]
    xr_ref[...] = lax.dot_general(
        x, wr_ref[...], (((1,), (0,)), ((), ())),
        precision=lax.Precision.HIGHEST,
        preferred_element_type=jnp.float32) + br_ref[...]


def _project(x, ws, wr, bs, br):
    blk = 1000
    grid = N // blk
    return pl.pallas_call(
        _proj_body,
        grid=(grid,),
        in_specs=[
            pl.BlockSpec((blk, D), lambda i: (i, 0)),
            pl.BlockSpec((D, D), lambda i: (0, 0)),
            pl.BlockSpec((D, D), lambda i: (0, 0)),
            pl.BlockSpec((1, D), lambda i: (0, 0)),
            pl.BlockSpec((1, D), lambda i: (0, 0)),
        ],
        out_specs=[
            pl.BlockSpec((blk, D), lambda i: (i, 0)),
            pl.BlockSpec((blk, D), lambda i: (i, 0)),
        ],
        out_shape=[
            jax.ShapeDtypeStruct((N, D), jnp.float32),
            jax.ShapeDtypeStruct((N, D), jnp.float32),
        ],
    )(x, ws, wr, bs, br)


def _sc_body(xs_hbm, xr_hbm, snd_hbm, rcv_hbm, av_hbm, zz_hbm, out_hbm,
             agg_sh, idx_s0, idx_r0, idx_c0, s_v0, r_v0, m_v0,
             idx_s1, idx_r1, idx_c1, s_v1, r_v1, m_v1, a_v,
             si_s0, si_r0, si_s1, si_r1, si_c0, si_c1,
             sg_s0, sg_r0, sg_s1, sg_r1, sc0, sc1):
    cid = lax.axis_index("c")
    sid = lax.axis_index("s")
    wid = sid * NC + cid

    idx_s = (idx_s0, idx_s1)
    idx_r = (idx_r0, idx_r1)
    idx_c = (idx_c0, idx_c1)
    s_v = (s_v0, s_v1)
    r_v = (r_v0, r_v1)
    m_v = (m_v0, m_v1)
    si_s = (si_s0, si_s1)
    si_r = (si_r0, si_r1)
    si_c = (si_c0, si_c1)
    sg_s = (sg_s0, sg_s1)
    sg_r = (sg_r0, sg_r1)
    sc = (sc0, sc1)

    pltpu.sync_copy(zz_hbm.at[pl.ds(sid * RPT, RPT)],
                    agg_sh.at[pl.ds(sid * RPT, RPT)])

    pltpu.sync_copy(av_hbm, a_v)
    plsc.subcore_barrier()

    a_vec = a_v[...]
    iota = lax.iota(jnp.int32, L)
    head_mask = [iota == h for h in range(H)]

    def issue_idx(win, b):
        e0 = wid * EPW + win * W
        pltpu.async_copy(snd_hbm.at[pl.ds(e0, W)], idx_s[b], si_s[b])
        pltpu.async_copy(rcv_hbm.at[pl.ds(e0, W)], idx_r[b], si_r[b])

    def wait_idx(b):
        pltpu.make_async_copy(snd_hbm.at[pl.ds(0, W)], idx_s[b], si_s[b]).wait()
        pltpu.make_async_copy(rcv_hbm.at[pl.ds(0, W)], idx_r[b], si_r[b]).wait()

    def issue_gather(b):
        pltpu.async_copy(xs_hbm.at[idx_s[b]], s_v[b], sg_s[b])
        pltpu.async_copy(xr_hbm.at[idx_r[b]], r_v[b], sg_r[b])

    def wait_gather(b):
        pltpu.make_async_copy(xs_hbm.at[idx_s[b]], s_v[b], sg_s[b]).wait()
        pltpu.make_async_copy(xr_hbm.at[idx_r[b]], r_v[b], sg_r[b]).wait()

    def wait_scatter(b):
        pltpu.make_async_copy(m_v[b], agg_sh.at[idx_c[b]], sc[b]).wait()

    e0 = wid * EPW
    pltpu.sync_copy(snd_hbm.at[pl.ds(e0, W)], idx_s[0])
    pltpu.sync_copy(rcv_hbm.at[pl.ds(e0, W)], idx_r[0])
    issue_gather(0)
    issue_idx(1, 1)

    @pl.loop(0, NWIN, step=2)
    def _(base):
        for b in (0, 1):
            win = base + b
            nb = 1 - b
            wait_idx(nb)
            issue_gather(nb)
            wait_gather(b)
            @pl.when(win >= 2)
            def _():
                wait_scatter(b)
            ec = wid * EPW + win * W
            pltpu.async_copy(rcv_hbm.at[pl.ds(ec, W)], idx_c[b], si_c[b])
            nxt = jnp.minimum(win + 2, NWIN - 1)
            issue_idx(nxt, b)

            @plsc.parallel_loop(0, W, unroll=2)
            def _(w):
                ss = [s_v[b][w, pl.ds(h * HD, HD)] for h in range(H)]
                rr = [r_v[b][w, pl.ds(h * HD, HD)] for h in range(H)]
                zz_ = [ss[h] + rr[h] for h in range(H)]
                uu = [jnp.exp(jnp.minimum(z, 20.0)) for z in zz_]
                tt = [u * (u + 2.0) for u in uu]
                pp = [(zz_[h] * a_vec) * tt[h] / (tt[h] + 2.0)
                      for h in range(H)]
                lg = [jnp.sum(p) for p in pp]
                uv = [jnp.exp(lax.broadcast(g, (L,))) for g in lg]
                du = jnp.zeros((L,), jnp.float32)
                for h in range(H):
                    m_v[b][w, pl.ds(h * HD, HD)] = ss[h] * uv[h]
                    du = du + jnp.where(head_mask[h], uv[h], 0.0)
                m_v[b][w, pl.ds(D, L)] = du

            pltpu.make_async_copy(rcv_hbm.at[pl.ds(0, W)], idx_c[b],
                                  si_c[b]).wait()
            pltpu.async_copy(m_v[b], agg_sh.at[idx_c[b]], sc[b], add=True)

    wait_scatter(0)
    wait_scatter(1)
    wait_gather(0)
    wait_idx(1)

    plsc.subcore_barrier()

    pltpu.sync_copy(agg_sh.at[pl.ds(sid * RPT, RPT)],
                    out_hbm.at[cid, pl.ds(sid * RPT, RPT)])


def _sc_edge_pass(xs, xr, snd, rcv, a_vec):
    mesh = plsc.VectorSubcoreMesh(core_axis_name="c", subcore_axis_name="s")
    cp = pltpu.CompilerParams()
    if "needs_layout_passes" in pltpu.CompilerParams.__dataclass_fields__:
        cp = dataclasses.replace(cp, needs_layout_passes=False)
    if "use_tc_tiling_on_sc" in pltpu.CompilerParams.__dataclass_fields__:
        cp = dataclasses.replace(cp, use_tc_tiling_on_sc=False)
    kern = pl.kernel(
        _sc_body,
        compiler_params=cp,
        out_type=jax.ShapeDtypeStruct((NC, N, MROW), jnp.float32),
        mesh=mesh,
        scratch_types=(
            [pltpu.VMEM_SHARED((N, MROW), jnp.float32)]
            + 2 * [
                pltpu.VMEM((W,), jnp.int32),
                pltpu.VMEM((W,), jnp.int32),
                pltpu.VMEM((W,), jnp.int32),
                pltpu.VMEM((W, D), jnp.float32),
                pltpu.VMEM((W, D), jnp.float32),
                pltpu.VMEM((W, MROW), jnp.float32),
            ]
            + [
                pltpu.VMEM((L,), jnp.float32),
            ]
            + 12 * [pltpu.SemaphoreType.DMA]
        ),
    )
    zz = jnp.zeros((N, MROW), jnp.float32)
    return kern(xs, xr, snd, rcv, a_vec, zz)


def _comb_body(p0_ref, p1_ref, o_ref):
    a = p0_ref[:, :D] + p1_ref[:, :D]
    dnm = p0_ref[:, D:D + H] + p1_ref[:, D:D + H]
    drep = jnp.concatenate(
        [jnp.broadcast_to(dnm[:, h:h + 1], (dnm.shape[0], HD))
         for h in range(H)], axis=1)
    o_ref[...] = jnp.where(drep > 0.0, a / drep, 0.0)


def _combine(partials):
    blk = 1000
    grid = N // blk
    p0 = partials[0]
    p1 = partials[1]
    return pl.pallas_call(
        _comb_body,
        grid=(grid,),
        in_specs=[
            pl.BlockSpec((blk, MROW), lambda i: (i, 0)),
            pl.BlockSpec((blk, MROW), lambda i: (i, 0)),
        ],
        out_specs=pl.BlockSpec((blk, D), lambda i: (i, 0)),
        out_shape=jax.ShapeDtypeStruct((N, D), jnp.float32),
    )(p0, p1)


def kernel(x, edge_index, Ws_k, Ws_b, Wr_k, Wr_b, A_k, A_b):
    ws = Ws_k.reshape(D, H * HD)
    wr = Wr_k.reshape(D, H * HD)
    bs = Ws_b.reshape(1, H * HD)
    br = Wr_b.reshape(1, H * HD)
    a_vec = A_k.reshape(HD)
    snd = edge_index[0]
    rcv = edge_index[1]

    xs, xr = _project(x, ws, wr, bs, br)
    partials = _sc_edge_pass(xs, xr, snd, rcv, a_vec)
    return _combine(partials)
